# baseline (speedup 1.0000x reference)
import jax
import jax.numpy as jnp
from jax import lax
from jax.experimental import pallas as pl
from jax.experimental.pallas import tpu as pltpu
from jax.experimental.pallas import tpu_sc as plsc

_NUM_NODES = 2000000
_LR_SCHEDULE = (0.1, 0.095, 0.09, 0.085, 0.08, 0.075, 0.07, 0.065, 0.06, 0.055)
_B = 1048576
_NC = 2
_NS = 16
_NW = _NC * _NS
_C = 8192
_G = 128
_N_W = _B // _NW


def _stress_body(i_hbm, j_hbm, vi_hbm, vj_hbm, dis_hbm, lr_hbm, pos_hbm,
                 out_hbm,
                 iv, jv, viv, vjv, disv,
                 idx_xi, idx_yi, idx_xj, idx_yj,
                 x_i, y_i, x_j, y_j,
                 lrv, accv, sem):
  wid = lax.axis_index("s") * _NC + lax.axis_index("c")
  pltpu.sync_copy(lr_hbm, lrv)
  accv[...] = jnp.zeros((16,), jnp.float32)

  def chunk_body(c, carry):
    base = wid * _N_W + c * _C
    copies = [
        pltpu.async_copy(i_hbm.at[pl.ds(base, _C)], iv, sem),
        pltpu.async_copy(j_hbm.at[pl.ds(base, _C)], jv, sem),
        pltpu.async_copy(vi_hbm.at[pl.ds(base, _C)], viv, sem),
        pltpu.async_copy(vj_hbm.at[pl.ds(base, _C)], vjv, sem),
        pltpu.async_copy(dis_hbm.at[pl.ds(base, _C)], disv, sem),
    ]
    for h in copies:
      h.wait()

    def idx_body(k, carry2):
      o = k * 16
      ei = (iv[pl.ds(o, 16)] - 1) * 2 + viv[pl.ds(o, 16)]
      ej = (jv[pl.ds(o, 16)] - 1) * 2 + vjv[pl.ds(o, 16)]
      ei = jnp.where(ei < 0, ei + _NUM_NODES, ei)
      ej = jnp.where(ej < 0, ej + _NUM_NODES, ej)
      xi = ei * 2
      xj = ej * 2
      idx_xi[pl.ds(o, 16)] = xi
      idx_yi[pl.ds(o, 16)] = xi + 1
      idx_xj[pl.ds(o, 16)] = xj
      idx_yj[pl.ds(o, 16)] = xj + 1
      return carry2

    lax.fori_loop(0, _C // 16, idx_body, 0)

    def gather_body(g, carry2):
      o = g * _G
      pltpu.async_copy(pos_hbm.at[idx_xi.at[pl.ds(o, _G)]],
                       x_i.at[pl.ds(o, _G)], sem)
      pltpu.async_copy(pos_hbm.at[idx_yi.at[pl.ds(o, _G)]],
                       y_i.at[pl.ds(o, _G)], sem)
      pltpu.async_copy(pos_hbm.at[idx_xj.at[pl.ds(o, _G)]],
                       x_j.at[pl.ds(o, _G)], sem)
      pltpu.async_copy(pos_hbm.at[idx_yj.at[pl.ds(o, _G)]],
                       y_j.at[pl.ds(o, _G)], sem)
      return carry2

    lax.fori_loop(0, _C // _G, gather_body, 0)
    for buf in (x_i, y_i, x_j, y_j):
      pltpu.make_async_copy(pos_hbm.at[pl.ds(0, _C)], buf, sem).wait()

    lrvec = lrv[...]

    def pair_body(k, carry2):
      o = k * 16
      dd = disv[pl.ds(o, 16)]
      dx = x_i[pl.ds(o, 16)] - x_j[pl.ds(o, 16)]
      dy = y_i[pl.ds(o, 16)] - y_j[pl.ds(o, 16)]
      d2 = jnp.maximum(dx * dx + dy * dy, 1e-30)
      bits = lax.bitcast_convert_type(d2, jnp.int32)
      r = lax.bitcast_convert_type(
          0x5F3759DF - lax.shift_right_arithmetic(bits, 1), jnp.float32)
      r = r * (1.5 - 0.5 * d2 * r * r)
      r = r * (1.5 - 0.5 * d2 * r * r)
      r = r * (1.5 - 0.5 * d2 * r * r)
      mag = d2 * r
      coeff = 0.25 / jnp.maximum(dd, lrvec)
      e = mag - dd
      accv[...] = accv[...] + coeff * e * e
      return carry2

    lax.fori_loop(0, _C // 16, pair_body, 0)
    return carry

  lax.fori_loop(0, _N_W // _C, chunk_body, 0)
  pltpu.sync_copy(accv, out_hbm.at[wid])


_mesh = plsc.VectorSubcoreMesh(core_axis_name="c", subcore_axis_name="s")
_call = pl.kernel(
    _stress_body,
    mesh=_mesh,
    out_type=jax.ShapeDtypeStruct((_NW, 16), jnp.float32),
    scratch_types=[
        pltpu.VMEM((_C,), jnp.int32),
        pltpu.VMEM((_C,), jnp.int32),
        pltpu.VMEM((_C,), jnp.int32),
        pltpu.VMEM((_C,), jnp.int32),
        pltpu.VMEM((_C,), jnp.float32),
        pltpu.VMEM((_C,), jnp.int32),
        pltpu.VMEM((_C,), jnp.int32),
        pltpu.VMEM((_C,), jnp.int32),
        pltpu.VMEM((_C,), jnp.int32),
        pltpu.VMEM((_C,), jnp.float32),
        pltpu.VMEM((_C,), jnp.float32),
        pltpu.VMEM((_C,), jnp.float32),
        pltpu.VMEM((_C,), jnp.float32),
        pltpu.VMEM((16,), jnp.float32),
        pltpu.VMEM((16,), jnp.float32),
        pltpu.SemaphoreType.DMA,
    ],
)


def kernel(i, j, vis_p_i, vis_p_j, dis, it, pos):
  lr = jnp.asarray(_LR_SCHEDULE, jnp.float32)[it]
  lr_vec = jnp.full((16,), lr, jnp.float32)
  out = _call(i.astype(jnp.int32), j.astype(jnp.int32),
              vis_p_i.astype(jnp.int32), vis_p_j.astype(jnp.int32),
              dis, lr_vec, pos.reshape(-1))
  return jnp.sum(out)


# speedup vs baseline: 9.9777x; 9.9777x over previous
import jax
import jax.numpy as jnp
from jax import lax
from jax.experimental import pallas as pl
from jax.experimental.pallas import tpu as pltpu
from jax.experimental.pallas import tpu_sc as plsc

_NUM_NODES = 2000000
_LR_SCHEDULE = (0.1, 0.095, 0.09, 0.085, 0.08, 0.075, 0.07, 0.065, 0.06, 0.055)
_B = 1048576
_NC = 2
_NS = 16
_NW = _NC * _NS
_C = 8192
_G = 128
_N_W = _B // _NW


def _stress_body(i_hbm, j_hbm, vi_hbm, vj_hbm, dis_hbm, lr_hbm, pos_hbm,
                 out_hbm,
                 iv, jv, viv, vjv, disv,
                 idx_xi, idx_yi, idx_xj, idx_yj, x_i, y_i, x_j, y_j,
                 lrv, accv, sem):
  wid = lax.axis_index("s") * _NC + lax.axis_index("c")
  pltpu.sync_copy(lr_hbm, lrv)
  accv[...] = jnp.zeros((16,), jnp.float32)

  def chunk_body(c, carry):
    base = wid * _N_W + c * _C
    copies = [
        pltpu.async_copy(i_hbm.at[pl.ds(base, _C)], iv, sem),
        pltpu.async_copy(j_hbm.at[pl.ds(base, _C)], jv, sem),
        pltpu.async_copy(vi_hbm.at[pl.ds(base, _C)], viv, sem),
        pltpu.async_copy(vj_hbm.at[pl.ds(base, _C)], vjv, sem),
        pltpu.async_copy(dis_hbm.at[pl.ds(base, _C)], disv, sem),
    ]
    for h in copies:
      h.wait()

    def idx_body(k, carry2):
      o = k * 16
      ei = (iv[pl.ds(o, 16)] - 1) * 2 + viv[pl.ds(o, 16)]
      ej = (jv[pl.ds(o, 16)] - 1) * 2 + vjv[pl.ds(o, 16)]
      ei = jnp.where(ei < 0, ei + _NUM_NODES, ei)
      ej = jnp.where(ej < 0, ej + _NUM_NODES, ej)
      idx_xi[pl.ds(o, 16)] = ei
      idx_yi[pl.ds(o, 16)] = ei + _NUM_NODES
      idx_xj[pl.ds(o, 16)] = ej
      idx_yj[pl.ds(o, 16)] = ej + _NUM_NODES
      return carry2

    lax.fori_loop(0, _C // 16, idx_body, 0)

    def gather_body(g, carry2):
      o = g * _G
      pltpu.async_copy(pos_hbm.at[idx_xi.at[pl.ds(o, _G)]],
                       x_i.at[pl.ds(o, _G)], sem)
      pltpu.async_copy(pos_hbm.at[idx_yi.at[pl.ds(o, _G)]],
                       y_i.at[pl.ds(o, _G)], sem)
      pltpu.async_copy(pos_hbm.at[idx_xj.at[pl.ds(o, _G)]],
                       x_j.at[pl.ds(o, _G)], sem)
      pltpu.async_copy(pos_hbm.at[idx_yj.at[pl.ds(o, _G)]],
                       y_j.at[pl.ds(o, _G)], sem)
      return carry2

    lax.fori_loop(0, _C // _G, gather_body, 0)
    for buf in (x_i, y_i, x_j, y_j):
      pltpu.make_async_copy(pos_hbm.at[pl.ds(0, _C)], buf, sem).wait()

    lrvec = lrv[...]

    def pair_body(k, carry2):
      o = k * 16
      dd = disv[pl.ds(o, 16)]
      dx = x_i[pl.ds(o, 16)] - x_j[pl.ds(o, 16)]
      dy = y_i[pl.ds(o, 16)] - y_j[pl.ds(o, 16)]
      d2 = jnp.maximum(dx * dx + dy * dy, 1e-30)
      bits = lax.bitcast_convert_type(d2, jnp.int32)
      r = lax.bitcast_convert_type(
          0x5F3759DF - lax.shift_right_arithmetic(bits, 1), jnp.float32)
      r = r * (1.5 - 0.5 * d2 * r * r)
      r = r * (1.5 - 0.5 * d2 * r * r)
      r = r * (1.5 - 0.5 * d2 * r * r)
      mag = d2 * r
      coeff = 0.25 / jnp.maximum(dd, lrvec)
      e = mag - dd
      accv[...] = accv[...] + coeff * e * e
      return carry2

    lax.fori_loop(0, _C // 16, pair_body, 0)
    return carry

  lax.fori_loop(0, _N_W // _C, chunk_body, 0)
  pltpu.sync_copy(accv, out_hbm.at[wid])


_mesh = plsc.VectorSubcoreMesh(core_axis_name="c", subcore_axis_name="s")
_call = pl.kernel(
    _stress_body,
    mesh=_mesh,
    out_type=jax.ShapeDtypeStruct((_NW, 16), jnp.float32),
    scratch_types=[
        pltpu.VMEM((_C,), jnp.int32),
        pltpu.VMEM((_C,), jnp.int32),
        pltpu.VMEM((_C,), jnp.int32),
        pltpu.VMEM((_C,), jnp.int32),
        pltpu.VMEM((_C,), jnp.float32),
        pltpu.VMEM((_C,), jnp.int32),
        pltpu.VMEM((_C,), jnp.int32),
        pltpu.VMEM((_C,), jnp.int32),
        pltpu.VMEM((_C,), jnp.int32),
        pltpu.VMEM((_C,), jnp.float32),
        pltpu.VMEM((_C,), jnp.float32),
        pltpu.VMEM((_C,), jnp.float32),
        pltpu.VMEM((_C,), jnp.float32),
        pltpu.VMEM((16,), jnp.float32),
        pltpu.VMEM((16,), jnp.float32),
        pltpu.SemaphoreType.DMA,
    ],
)


def kernel(i, j, vis_p_i, vis_p_j, dis, it, pos):
  lr = jnp.asarray(_LR_SCHEDULE, jnp.float32)[it]
  lr_vec = jnp.full((16,), lr, jnp.float32)
  out = _call(i.astype(jnp.int32), j.astype(jnp.int32),
              vis_p_i.astype(jnp.int32), vis_p_j.astype(jnp.int32),
              dis, lr_vec, pos.T.reshape(-1))
  return jnp.sum(out)


# speedup vs baseline: 10.8492x; 1.0873x over previous
import jax
import jax.numpy as jnp
from jax import lax
from jax.experimental import pallas as pl
from jax.experimental.pallas import tpu as pltpu
from jax.experimental.pallas import tpu_sc as plsc

_NUM_NODES = 2000000
_LR_SCHEDULE = (0.1, 0.095, 0.09, 0.085, 0.08, 0.075, 0.07, 0.065, 0.06, 0.055)
_B = 1048576
_NC = 2
_NS = 16
_NW = _NC * _NS
_C = 4096
_G = 128
_N_W = _B // _NW
_CHUNKS = _N_W // _C


def _stress_body(i_hbm, j_hbm, vi_hbm, vj_hbm, dis_hbm, lr_hbm, pos_hbm,
                 out_hbm,
                 iv0, jv0, viv0, vjv0, disv0,
                 iv1, jv1, viv1, vjv1, disv1,
                 idx_xi, idx_yi, idx_xj, idx_yj,
                 xi0, yi0, xj0, yj0,
                 xi1, yi1, xj1, yj1,
                 lrv, accv, sem_in, sem_g):
  wid = lax.axis_index("s") * _NC + lax.axis_index("c")
  ins = ((iv0, jv0, viv0, vjv0, disv0), (iv1, jv1, viv1, vjv1, disv1))
  xys = ((xi0, yi0, xj0, yj0), (xi1, yi1, xj1, yj1))
  pltpu.sync_copy(lr_hbm, lrv)
  accv[...] = jnp.zeros((16,), jnp.float32)
  lrvec = lrv[...]

  def issue_inputs(c, s):
    base = wid * _N_W + c * _C
    for src, dst in zip((i_hbm, j_hbm, vi_hbm, vj_hbm, dis_hbm), ins[s]):
      pltpu.async_copy(src.at[pl.ds(base, _C)], dst, sem_in)

  def drain_inputs(s):
    for src, dst in zip((i_hbm, j_hbm, vi_hbm, vj_hbm, dis_hbm), ins[s]):
      pltpu.make_async_copy(src.at[pl.ds(0, _C)], dst, sem_in).wait()

  def idx_compute(s):
    iv, jv, viv, vjv, _ = ins[s]

    def idx_body(k, carry):
      o = k * 16
      ei = (iv[pl.ds(o, 16)] - 1) * 2 + viv[pl.ds(o, 16)]
      ej = (jv[pl.ds(o, 16)] - 1) * 2 + vjv[pl.ds(o, 16)]
      ei = jnp.where(ei < 0, ei + _NUM_NODES, ei)
      ej = jnp.where(ej < 0, ej + _NUM_NODES, ej)
      idx_xi[pl.ds(o, 16)] = ei
      idx_yi[pl.ds(o, 16)] = ei + _NUM_NODES
      idx_xj[pl.ds(o, 16)] = ej
      idx_yj[pl.ds(o, 16)] = ej + _NUM_NODES
      return carry

    lax.fori_loop(0, _C // 16, idx_body, 0, unroll=4)

  def issue_gathers(s):
    x_i, y_i, x_j, y_j = xys[s]

    def gather_body(g, carry):
      o = g * _G
      pltpu.async_copy(pos_hbm.at[idx_xi.at[pl.ds(o, _G)]],
                       x_i.at[pl.ds(o, _G)], sem_g)
      pltpu.async_copy(pos_hbm.at[idx_yi.at[pl.ds(o, _G)]],
                       y_i.at[pl.ds(o, _G)], sem_g)
      pltpu.async_copy(pos_hbm.at[idx_xj.at[pl.ds(o, _G)]],
                       x_j.at[pl.ds(o, _G)], sem_g)
      pltpu.async_copy(pos_hbm.at[idx_yj.at[pl.ds(o, _G)]],
                       y_j.at[pl.ds(o, _G)], sem_g)
      return carry

    lax.fori_loop(0, _C // _G, gather_body, 0)

  def drain_gathers(s):
    for buf in xys[s]:
      pltpu.make_async_copy(pos_hbm.at[pl.ds(0, _C)], buf, sem_g).wait()

  def pair_compute(s):
    x_i, y_i, x_j, y_j = xys[s]
    disv = ins[s][4]

    def pair_body(k, carry):
      o = k * 16
      dd = disv[pl.ds(o, 16)]
      dx = x_i[pl.ds(o, 16)] - x_j[pl.ds(o, 16)]
      dy = y_i[pl.ds(o, 16)] - y_j[pl.ds(o, 16)]
      d2 = jnp.maximum(dx * dx + dy * dy, 1e-30)
      bits = lax.bitcast_convert_type(d2, jnp.int32)
      r = lax.bitcast_convert_type(
          0x5F3759DF - lax.shift_right_arithmetic(bits, 1), jnp.float32)
      r = r * (1.5 - 0.5 * d2 * r * r)
      r = r * (1.5 - 0.5 * d2 * r * r)
      r = r * (1.5 - 0.5 * d2 * r * r)
      mag = d2 * r
      coeff = 0.25 / jnp.maximum(dd, lrvec)
      e = mag - dd
      accv[...] = accv[...] + coeff * e * e
      return carry

    lax.fori_loop(0, _C // 16, pair_body, 0, unroll=4)

  issue_inputs(0, 0)
  for c in range(_CHUNKS):
    s = c % 2
    drain_inputs(s)
    idx_compute(s)
    issue_gathers(s)
    if c > 0:
      pair_compute(1 - s)
    if c + 1 < _CHUNKS:
      issue_inputs(c + 1, 1 - s)
    drain_gathers(s)
  pair_compute((_CHUNKS - 1) % 2)
  pltpu.sync_copy(accv, out_hbm.at[wid])


_mesh = plsc.VectorSubcoreMesh(core_axis_name="c", subcore_axis_name="s")
_scratch = (
    [pltpu.VMEM((_C,), jnp.int32)] * 4 + [pltpu.VMEM((_C,), jnp.float32)]
) * 2 + [pltpu.VMEM((_C,), jnp.int32)] * 4 + [
    pltpu.VMEM((_C,), jnp.float32)
] * 8 + [
    pltpu.VMEM((16,), jnp.float32),
    pltpu.VMEM((16,), jnp.float32),
    pltpu.SemaphoreType.DMA,
    pltpu.SemaphoreType.DMA,
]
_call = pl.kernel(
    _stress_body,
    mesh=_mesh,
    out_type=jax.ShapeDtypeStruct((_NW, 16), jnp.float32),
    scratch_types=_scratch,
)


def kernel(i, j, vis_p_i, vis_p_j, dis, it, pos):
  lr = jnp.asarray(_LR_SCHEDULE, jnp.float32)[it]
  lr_vec = jnp.full((16,), lr, jnp.float32)
  out = _call(i.astype(jnp.int32), j.astype(jnp.int32),
              vis_p_i.astype(jnp.int32), vis_p_j.astype(jnp.int32),
              dis, lr_vec, pos.T.reshape(-1))
  return jnp.sum(out)


# speedup vs baseline: 12.6270x; 1.1639x over previous
import jax
import jax.numpy as jnp
from jax import lax
from jax.experimental import pallas as pl
from jax.experimental.pallas import tpu as pltpu
from jax.experimental.pallas import tpu_sc as plsc

_NUM_NODES = 2000000
_LR_SCHEDULE = (0.1, 0.095, 0.09, 0.085, 0.08, 0.075, 0.07, 0.065, 0.06, 0.055)
_B = 1048576
_NC = 2
_NS = 16
_NW = _NC * _NS
_C = 4096
_G = 128
_N_W = _B // _NW
_CHUNKS = _N_W // _C


def _stress_body(i_hbm, j_hbm, vi_hbm, vj_hbm, dis_hbm, lr_hbm, pos_hbm,
                 out_hbm,
                 iv0, jv0, viv0, vjv0, disv0,
                 iv1, jv1, viv1, vjv1, disv1,
                 idx_xi, idx_yi, idx_xj, idx_yj,
                 xi0, yi0, xj0, yj0,
                 xi1, yi1, xj1, yj1,
                 lrv, accv, sem_in, sem_g):
  wid = lax.axis_index("s") * _NC + lax.axis_index("c")
  ins = ((iv0, jv0, viv0, vjv0, disv0), (iv1, jv1, viv1, vjv1, disv1))
  xys = ((xi0, yi0, xj0, yj0), (xi1, yi1, xj1, yj1))
  pltpu.sync_copy(lr_hbm, lrv)
  accv[...] = jnp.zeros((16,), jnp.float32)
  lrvec = lrv[...]

  def issue_inputs(c, s):
    base = wid * _N_W + c * _C
    for src, dst in zip((i_hbm, j_hbm, vi_hbm, vj_hbm, dis_hbm), ins[s]):
      pltpu.async_copy(src.at[pl.ds(base, _C)], dst, sem_in)

  def drain_inputs(s):
    for src, dst in zip((i_hbm, j_hbm, vi_hbm, vj_hbm, dis_hbm), ins[s]):
      pltpu.make_async_copy(src.at[pl.ds(0, _C)], dst, sem_in).wait()

  def idx_compute(s):
    iv, jv, viv, vjv, _ = ins[s]

    @plsc.parallel_loop(0, _C, step=16, unroll=4)
    def _idx_body(o):
      ei = (iv[pl.ds(o, 16)] - 1) * 2 + viv[pl.ds(o, 16)]
      ej = (jv[pl.ds(o, 16)] - 1) * 2 + vjv[pl.ds(o, 16)]
      ei = jnp.where(ei < 0, ei + _NUM_NODES, ei)
      ej = jnp.where(ej < 0, ej + _NUM_NODES, ej)
      idx_xi[pl.ds(o, 16)] = ei
      idx_yi[pl.ds(o, 16)] = ei + _NUM_NODES
      idx_xj[pl.ds(o, 16)] = ej
      idx_yj[pl.ds(o, 16)] = ej + _NUM_NODES

  def issue_gathers(s):
    x_i, y_i, x_j, y_j = xys[s]

    def gather_body(g, carry):
      o = g * _G
      pltpu.async_copy(pos_hbm.at[idx_xi.at[pl.ds(o, _G)]],
                       x_i.at[pl.ds(o, _G)], sem_g)
      pltpu.async_copy(pos_hbm.at[idx_yi.at[pl.ds(o, _G)]],
                       y_i.at[pl.ds(o, _G)], sem_g)
      pltpu.async_copy(pos_hbm.at[idx_xj.at[pl.ds(o, _G)]],
                       x_j.at[pl.ds(o, _G)], sem_g)
      pltpu.async_copy(pos_hbm.at[idx_yj.at[pl.ds(o, _G)]],
                       y_j.at[pl.ds(o, _G)], sem_g)
      return carry

    lax.fori_loop(0, _C // _G, gather_body, 0)

  def drain_gathers(s):
    for buf in xys[s]:
      pltpu.make_async_copy(pos_hbm.at[pl.ds(0, _C)], buf, sem_g).wait()

  def pair_compute(s):
    x_i, y_i, x_j, y_j = xys[s]
    disv = ins[s][4]

    @plsc.parallel_loop(0, _C, step=16, unroll=8,
                        carry=jnp.zeros((16,), jnp.float32))
    def acc(o, a):
      dd = disv[pl.ds(o, 16)]
      dx = x_i[pl.ds(o, 16)] - x_j[pl.ds(o, 16)]
      dy = y_i[pl.ds(o, 16)] - y_j[pl.ds(o, 16)]
      d2 = jnp.maximum(dx * dx + dy * dy, 1e-30)
      bits = lax.bitcast_convert_type(d2, jnp.int32)
      r = lax.bitcast_convert_type(
          0x5F3759DF - lax.shift_right_arithmetic(bits, 1), jnp.float32)
      r = r * (1.5 - 0.5 * d2 * r * r)
      r = r * (1.5 - 0.5 * d2 * r * r)
      mag = d2 * r
      coeff = 0.25 / jnp.maximum(dd, lrvec)
      e = mag - dd
      return a + coeff * e * e

    accv[...] = accv[...] + acc

  issue_inputs(0, 0)
  for c in range(_CHUNKS):
    s = c % 2
    drain_inputs(s)
    idx_compute(s)
    issue_gathers(s)
    if c > 0:
      pair_compute(1 - s)
    if c + 1 < _CHUNKS:
      issue_inputs(c + 1, 1 - s)
    drain_gathers(s)
  pair_compute((_CHUNKS - 1) % 2)
  pltpu.sync_copy(accv, out_hbm.at[wid])


_mesh = plsc.VectorSubcoreMesh(core_axis_name="c", subcore_axis_name="s")
_scratch = (
    [pltpu.VMEM((_C,), jnp.int32)] * 4 + [pltpu.VMEM((_C,), jnp.float32)]
) * 2 + [pltpu.VMEM((_C,), jnp.int32)] * 4 + [
    pltpu.VMEM((_C,), jnp.float32)
] * 8 + [
    pltpu.VMEM((16,), jnp.float32),
    pltpu.VMEM((16,), jnp.float32),
    pltpu.SemaphoreType.DMA,
    pltpu.SemaphoreType.DMA,
]
_call = pl.kernel(
    _stress_body,
    mesh=_mesh,
    out_type=jax.ShapeDtypeStruct((_NW, 16), jnp.float32),
    scratch_types=_scratch,
)


def kernel(i, j, vis_p_i, vis_p_j, dis, it, pos):
  lr = jnp.asarray(_LR_SCHEDULE, jnp.float32)[it]
  lr_vec = jnp.full((16,), lr, jnp.float32)
  out = _call(i.astype(jnp.int32), j.astype(jnp.int32),
              vis_p_i.astype(jnp.int32), vis_p_j.astype(jnp.int32),
              dis, lr_vec, pos.T.reshape(-1))
  return jnp.sum(out)


# speedup vs baseline: 13.8149x; 1.0941x over previous
import jax
import jax.numpy as jnp
from jax import lax
from jax.experimental import pallas as pl
from jax.experimental.pallas import tpu as pltpu
from jax.experimental.pallas import tpu_sc as plsc

_NUM_NODES = 2000000
_LR_SCHEDULE = (0.1, 0.095, 0.09, 0.085, 0.08, 0.075, 0.07, 0.065, 0.06, 0.055)
_B = 1048576
_NC = 2
_NS = 16
_NW = _NC * _NS
_C = 4096
_G = 128
_N_W = _B // _NW
_CHUNKS = _N_W // _C


def _stress_body(i_hbm, j_hbm, vi_hbm, vj_hbm, dis_hbm, lr_hbm, pos_hbm,
                 out_hbm,
                 iv0, jv0, viv0, vjv0, disv0,
                 iv1, jv1, viv1, vjv1, disv1,
                 idx_i, idx_j,
                 pi0, pj0, pi1, pj1,
                 lrv, accv, sem_in, sem_g):
  wid = lax.axis_index("s") * _NC + lax.axis_index("c")
  ins = ((iv0, jv0, viv0, vjv0, disv0), (iv1, jv1, viv1, vjv1, disv1))
  gbufs = ((pi0, pj0), (pi1, pj1))
  pltpu.sync_copy(lr_hbm, lrv)
  accv[...] = jnp.zeros((16,), jnp.float32)
  lrvec = lrv[...]

  def issue_inputs(c, s):
    base = wid * _N_W + c * _C
    for src, dst in zip((i_hbm, j_hbm, vi_hbm, vj_hbm, dis_hbm), ins[s]):
      pltpu.async_copy(src.at[pl.ds(base, _C)], dst, sem_in)

  def drain_inputs(s):
    for src, dst in zip((i_hbm, j_hbm, vi_hbm, vj_hbm, dis_hbm), ins[s]):
      pltpu.make_async_copy(src.at[pl.ds(0, _C)], dst, sem_in).wait()

  def idx_compute(s):
    iv, jv, viv, vjv, _ = ins[s]

    @plsc.parallel_loop(0, _C, step=16, unroll=4)
    def _idx_body(o):
      ei = (iv[pl.ds(o, 16)] - 1) * 2 + viv[pl.ds(o, 16)]
      ej = (jv[pl.ds(o, 16)] - 1) * 2 + vjv[pl.ds(o, 16)]
      idx_i[pl.ds(o, 16)] = jnp.where(ei < 0, ei + _NUM_NODES, ei)
      idx_j[pl.ds(o, 16)] = jnp.where(ej < 0, ej + _NUM_NODES, ej)

  def issue_gathers(s):
    p_i, p_j = gbufs[s]

    def gather_body(g, carry):
      o = g * _G
      pltpu.async_copy(pos_hbm.at[idx_i.at[pl.ds(o, _G)]],
                       p_i.at[pl.ds(o, _G)], sem_g)
      pltpu.async_copy(pos_hbm.at[idx_j.at[pl.ds(o, _G)]],
                       p_j.at[pl.ds(o, _G)], sem_g)
      return carry

    lax.fori_loop(0, _C // _G, gather_body, 0)

  def drain_gathers(s):
    for buf in gbufs[s]:
      pltpu.make_async_copy(pos_hbm.at[pl.ds(0, _C)], buf, sem_g).wait()

  def pair_compute(s):
    p_i, p_j = gbufs[s]
    disv = ins[s][4]
    hi_mask = jnp.full((16,), -65536, jnp.int32)

    @plsc.parallel_loop(0, _C, step=16, unroll=8,
                        carry=jnp.zeros((16,), jnp.float32))
    def acc(o, a):
      dd = disv[pl.ds(o, 16)]
      wi = p_i[pl.ds(o, 16)]
      wj = p_j[pl.ds(o, 16)]
      x_i = lax.bitcast_convert_type(lax.shift_left(wi, 16), jnp.float32)
      y_i = lax.bitcast_convert_type(wi & hi_mask, jnp.float32)
      x_j = lax.bitcast_convert_type(lax.shift_left(wj, 16), jnp.float32)
      y_j = lax.bitcast_convert_type(wj & hi_mask, jnp.float32)
      dx = x_i - x_j
      dy = y_i - y_j
      d2 = jnp.maximum(dx * dx + dy * dy, 1e-30)
      bits = lax.bitcast_convert_type(d2, jnp.int32)
      r = lax.bitcast_convert_type(
          0x5F3759DF - lax.shift_right_arithmetic(bits, 1), jnp.float32)
      r = r * (1.5 - 0.5 * d2 * r * r)
      r = r * (1.5 - 0.5 * d2 * r * r)
      mag = d2 * r
      coeff = 0.25 / jnp.maximum(dd, lrvec)
      e = mag - dd
      return a + coeff * e * e

    accv[...] = accv[...] + acc

  issue_inputs(0, 0)
  for c in range(_CHUNKS):
    s = c % 2
    drain_inputs(s)
    idx_compute(s)
    issue_gathers(s)
    if c > 0:
      pair_compute(1 - s)
    if c + 1 < _CHUNKS:
      issue_inputs(c + 1, 1 - s)
    drain_gathers(s)
  pair_compute((_CHUNKS - 1) % 2)
  pltpu.sync_copy(accv, out_hbm.at[wid])


_mesh = plsc.VectorSubcoreMesh(core_axis_name="c", subcore_axis_name="s")
_scratch = (
    [pltpu.VMEM((_C,), jnp.int32)] * 4 + [pltpu.VMEM((_C,), jnp.float32)]
) * 2 + [
    pltpu.VMEM((_C,), jnp.int32)
] * 2 + [
    pltpu.VMEM((_C,), jnp.int32)
] * 4 + [
    pltpu.VMEM((16,), jnp.float32),
    pltpu.VMEM((16,), jnp.float32),
    pltpu.SemaphoreType.DMA,
    pltpu.SemaphoreType.DMA,
]
_call = pl.kernel(
    _stress_body,
    mesh=_mesh,
    out_type=jax.ShapeDtypeStruct((_NW, 16), jnp.float32),
    scratch_types=_scratch,
)


def kernel(i, j, vis_p_i, vis_p_j, dis, it, pos):
  lr = jnp.asarray(_LR_SCHEDULE, jnp.float32)[it]
  lr_vec = jnp.full((16,), lr, jnp.float32)
  pos_packed = lax.bitcast_convert_type(
      pos.astype(jnp.bfloat16), jnp.int32)
  out = _call(i.astype(jnp.int32), j.astype(jnp.int32),
              vis_p_i.astype(jnp.int32), vis_p_j.astype(jnp.int32),
              dis, lr_vec, pos_packed)
  return jnp.sum(out)
